# baseline (speedup 1.0000x reference)
import jax
import jax.numpy as jnp
import numpy as np
from jax.experimental import pallas as pl
from jax.experimental.pallas import tpu as pltpu

N = 10000
E = 320000
FIN = 128
FE = 16
H = 4
C = 128
CE = 16
K = 5000

BN = 400
BE = 3200


def _node_proj_kernel(x_ref, wn_ref, bn_ref, h_ref):
    h_ref[...] = jnp.dot(x_ref[...], wn_ref[...],
                         preferred_element_type=jnp.float32) + bn_ref[...]


def _node_proj(x, W_node, b_node):
    return pl.pallas_call(
        _node_proj_kernel,
        grid=(N // BN,),
        in_specs=[
            pl.BlockSpec((BN, FIN), lambda i: (i, 0)),
            pl.BlockSpec((FIN, H * C), lambda i: (0, 0)),
            pl.BlockSpec((H * C,), lambda i: (0,)),
        ],
        out_specs=pl.BlockSpec((BN, H * C), lambda i: (i, 0)),
        out_shape=jax.ShapeDtypeStruct((N, H * C), jnp.float32),
    )(x, W_node, b_node)


def _edge_proj_kernel(ea_ref, we_ref, be_ref, e_ref):
    e_ref[...] = jnp.dot(ea_ref[...], we_ref[...],
                         preferred_element_type=jnp.float32) + be_ref[...]


def _edge_proj(edge_attr, W_edge, b_edge):
    return pl.pallas_call(
        _edge_proj_kernel,
        grid=(E // BE,),
        in_specs=[
            pl.BlockSpec((BE, FE), lambda i: (i, 0)),
            pl.BlockSpec((FE, H * CE), lambda i: (0, 0)),
            pl.BlockSpec((H * CE,), lambda i: (0,)),
        ],
        out_specs=pl.BlockSpec((BE, H * CE), lambda i: (i, 0)),
        out_shape=jax.ShapeDtypeStruct((E, H * CE), jnp.float32),
    )(edge_attr, W_edge, b_edge)


CHUNK = 256


def _aexp_kernel(al_ref, o_ref):
    al = al_ref[...]
    o_ref[...] = jnp.concatenate(
        [jnp.broadcast_to(al[:, hh:hh + 1], (BE, C)) for hh in range(H)], axis=1)


def _alpha_expand(alpha):
    return pl.pallas_call(
        _aexp_kernel,
        grid=(E // BE,),
        in_specs=[pl.BlockSpec((BE, H), lambda i: (i, 0))],
        out_specs=pl.BlockSpec((BE, H * C), lambda i: (i, 0)),
        out_shape=jax.ShapeDtypeStruct((E, H * C), jnp.float32),
    )(alpha)


def _agg_kernel(dst_ref, src_ref, aexp_ref, h_ref, out_ref):
    @pl.when(pl.program_id(0) == 0)
    def _init():
        out_ref[...] = jnp.zeros(out_ref.shape, out_ref.dtype)

    def body(i, carry):
        d = dst_ref[i]
        s = src_ref[i]
        prod = aexp_ref[pl.ds(i, 1), :] * h_ref[pl.ds(s, 1), :]
        out_ref[pl.ds(d, 1), :] += prod
        return carry

    jax.lax.fori_loop(0, CHUNK, body, 0)


def _aggregate(dstv, srcv, alpha, hflat):
    aexp = _alpha_expand(alpha)
    return pl.pallas_call(
        _agg_kernel,
        grid=(E // CHUNK,),
        in_specs=[
            pl.BlockSpec((CHUNK,), lambda i: (i,), memory_space=pltpu.SMEM),
            pl.BlockSpec((CHUNK,), lambda i: (i,), memory_space=pltpu.SMEM),
            pl.BlockSpec((CHUNK, H * C), lambda i: (i, 0)),
            pl.BlockSpec((N, H * C), lambda i: (0, 0)),
        ],
        out_specs=pl.BlockSpec((N, H * C), lambda i: (0, 0)),
        out_shape=jax.ShapeDtypeStruct((N, H * C), jnp.float32),
    )(dstv, srcv, aexp, hflat)


def kernel(x, edge_attr, edge_index, batch, W_node, b_node, W_edge, b_edge, att, p):
    src = edge_index[0]
    dst = edge_index[1]

    h = _node_proj(x, W_node, b_node).reshape(N, H, C)
    e = _edge_proj(edge_attr, W_edge, b_edge).reshape(E, H, CE)

    h_src = h[src]
    h_dst = h[dst]
    feat = jnp.concatenate([h_dst, h_src, e], axis=-1)
    logits = jax.nn.leaky_relu(jnp.einsum('ehf,hf->eh', feat, att), 0.2)
    m = jax.ops.segment_max(logits, dst, num_segments=N)
    m = jnp.where(jnp.isfinite(m), m, 0.0)
    ex = jnp.exp(logits - m[dst])
    den = jax.ops.segment_sum(ex, dst, num_segments=N)
    alpha = ex / (den[dst] + 1e-16)
    hflat = h.reshape(N, H * C)
    out = _aggregate(dst, src, alpha, hflat).reshape(N, H, C)
    nx = jax.nn.relu(out.mean(axis=1))
    ne = jax.nn.relu(e.mean(axis=1))

    score = jnp.tanh(nx @ p / (jnp.linalg.norm(p) + 1e-16))
    topv, perm = jax.lax.top_k(score, K)
    x_new = nx[perm] * topv[:, None]
    mask = jnp.zeros((N,), bool).at[perm].set(True)
    mapping = jnp.zeros((N,), jnp.int32).at[perm].set(jnp.arange(K, dtype=jnp.int32))
    emask = mask[src] & mask[dst]
    new_ei = jnp.where(emask[None, :], mapping[edge_index], -1)
    new_ea = jnp.where(emask[:, None], ne, 0.0)
    batch_new = batch[perm]
    return x_new, new_ei, new_ea, batch_new, perm, score


# speedup vs baseline: 1.2358x; 1.2358x over previous
import jax
import jax.numpy as jnp
import numpy as np
from jax.experimental import pallas as pl
from jax.experimental.pallas import tpu as pltpu

N = 10000
E = 320000
FIN = 128
FE = 16
H = 4
C = 128
CE = 16
K = 5000

BN = 400
BE = 3200


def _node_proj_kernel(x_ref, wn_ref, bn_ref, h_ref):
    h_ref[...] = jnp.dot(x_ref[...], wn_ref[...],
                         preferred_element_type=jnp.float32) + bn_ref[...]


def _node_proj(x, W_node, b_node):
    return pl.pallas_call(
        _node_proj_kernel,
        grid=(N // BN,),
        in_specs=[
            pl.BlockSpec((BN, FIN), lambda i: (i, 0)),
            pl.BlockSpec((FIN, H * C), lambda i: (0, 0)),
            pl.BlockSpec((H * C,), lambda i: (0,)),
        ],
        out_specs=pl.BlockSpec((BN, H * C), lambda i: (i, 0)),
        out_shape=jax.ShapeDtypeStruct((N, H * C), jnp.float32),
    )(x, W_node, b_node)


def _edge_proj_kernel(ea_ref, we_ref, be_ref, e_ref):
    e_ref[...] = jnp.dot(ea_ref[...], we_ref[...],
                         preferred_element_type=jnp.float32) + be_ref[...]


def _edge_proj(edge_attr, W_edge, b_edge):
    return pl.pallas_call(
        _edge_proj_kernel,
        grid=(E // BE,),
        in_specs=[
            pl.BlockSpec((BE, FE), lambda i: (i, 0)),
            pl.BlockSpec((FE, H * CE), lambda i: (0, 0)),
            pl.BlockSpec((H * CE,), lambda i: (0,)),
        ],
        out_specs=pl.BlockSpec((BE, H * CE), lambda i: (i, 0)),
        out_shape=jax.ShapeDtypeStruct((E, H * CE), jnp.float32),
    )(edge_attr, W_edge, b_edge)


CHUNK = 256


def _upd_kernel(hs_ref, al_ref, o_ref):
    al = al_ref[...]
    mult = jnp.concatenate(
        [jnp.broadcast_to(al[:, hh:hh + 1], (BE, C)) for hh in range(H)], axis=1)
    o_ref[...] = hs_ref[...] * mult


def _make_updates(hsflat, alpha):
    return pl.pallas_call(
        _upd_kernel,
        grid=(E // BE,),
        in_specs=[pl.BlockSpec((BE, H * C), lambda i: (i, 0)),
                  pl.BlockSpec((BE, H), lambda i: (i, 0))],
        out_specs=pl.BlockSpec((BE, H * C), lambda i: (i, 0)),
        out_shape=jax.ShapeDtypeStruct((E, H * C), jnp.float32),
    )(hsflat, alpha)


def _agg_kernel(dst_ref, upd_ref, out_ref):
    @pl.when(pl.program_id(0) == 0)
    def _init():
        out_ref[...] = jnp.zeros(out_ref.shape, out_ref.dtype)

    def body(i, carry):
        d = dst_ref[i]
        out_ref[pl.ds(d, 1), :] += upd_ref[pl.ds(i, 1), :]
        return carry

    jax.lax.fori_loop(0, CHUNK, body, 0)


def _aggregate(dstv, hsflat, alpha):
    upd = _make_updates(hsflat, alpha)
    return pl.pallas_call(
        _agg_kernel,
        grid=(E // CHUNK,),
        in_specs=[
            pl.BlockSpec((CHUNK,), lambda i: (i,), memory_space=pltpu.SMEM),
            pl.BlockSpec((CHUNK, H * C), lambda i: (i, 0)),
        ],
        out_specs=pl.BlockSpec((N, H * C), lambda i: (0, 0)),
        out_shape=jax.ShapeDtypeStruct((N, H * C), jnp.float32),
    )(dstv, upd)


def kernel(x, edge_attr, edge_index, batch, W_node, b_node, W_edge, b_edge, att, p):
    src = edge_index[0]
    dst = edge_index[1]

    h = _node_proj(x, W_node, b_node).reshape(N, H, C)
    e = _edge_proj(edge_attr, W_edge, b_edge).reshape(E, H, CE)

    h_src = h[src]
    h_dst = h[dst]
    feat = jnp.concatenate([h_dst, h_src, e], axis=-1)
    logits = jax.nn.leaky_relu(jnp.einsum('ehf,hf->eh', feat, att), 0.2)
    m = jax.ops.segment_max(logits, dst, num_segments=N)
    m = jnp.where(jnp.isfinite(m), m, 0.0)
    ex = jnp.exp(logits - m[dst])
    den = jax.ops.segment_sum(ex, dst, num_segments=N)
    alpha = ex / (den[dst] + 1e-16)
    out = _aggregate(dst, h_src.reshape(E, H * C), alpha).reshape(N, H, C)
    nx = jax.nn.relu(out.mean(axis=1))
    ne = jax.nn.relu(e.mean(axis=1))

    score = jnp.tanh(nx @ p / (jnp.linalg.norm(p) + 1e-16))
    topv, perm = jax.lax.top_k(score, K)
    x_new = nx[perm] * topv[:, None]
    mask = jnp.zeros((N,), bool).at[perm].set(True)
    mapping = jnp.zeros((N,), jnp.int32).at[perm].set(jnp.arange(K, dtype=jnp.int32))
    emask = mask[src] & mask[dst]
    new_ei = jnp.where(emask[None, :], mapping[edge_index], -1)
    new_ea = jnp.where(emask[:, None], ne, 0.0)
    batch_new = batch[perm]
    return x_new, new_ei, new_ea, batch_new, perm, score
